# Initial kernel scaffold; baseline (speedup 1.0000x reference)
import jax
import jax.numpy as jnp
from jax.experimental import pallas as pl


def kernel(agent_obs, hideout_obs, timestep_obs, num_agents, W_i, Wh_i, bh_i, b_i, wc_i, W_f, Wh_f, bh_f, b_f, wc_f, W_c, Wh_c, bh_c, b_c, W_o, Wh_o, bh_o, b_o, wc_o, Wl_ag, bl_ag, Wr_ag, Wl_hi, bl_hi, Wr_hi):
    raise NotImplementedError("write your pallas kernel here")


# speedup vs baseline: 2.8124x; 2.8124x over previous
import jax
import jax.numpy as jnp
from jax.experimental import pallas as pl
from jax.experimental.pallas import tpu as pltpu

_B, _S, _A, _F, _HID, _OUT = 256, 50, 16, 16, 64, 64
_N = _B * _A
_G4 = 4 * _HID


def _gclstm_kernel(x_ref, hide_ref, na_ref, W_ref, Wh_ref, b_ref,
                   wci_ref, wcf_ref, wco_ref, Wl_ref, Whi_ref, bl_ref,
                   out_ref, H_ref, C_ref):
    H_ref[...] = jnp.zeros((_HID, _N), jnp.float32)
    C_ref[...] = jnp.zeros((_HID, _N), jnp.float32)
    W = W_ref[...]
    Wh = Wh_ref[...]
    b = b_ref[...]
    wci = wci_ref[...]
    wcf = wcf_ref[...]
    wco = wco_ref[...]

    def step(t, carry):
        x = x_ref[t]
        H = H_ref[...]
        C = C_ref[...]
        G = (jnp.dot(W, x, preferred_element_type=jnp.float32)
             + jnp.dot(Wh, H, preferred_element_type=jnp.float32) + b)
        gi = jax.nn.sigmoid(G[0:_HID] + wci * C)
        gf = jax.nn.sigmoid(G[_HID:2 * _HID] + wcf * C)
        gt = jnp.tanh(G[2 * _HID:3 * _HID])
        Cn = gf * C + gi * gt
        go = jax.nn.sigmoid(G[3 * _HID:4 * _HID] + wco * Cn)
        H_ref[...] = go * jnp.tanh(Cn)
        C_ref[...] = Cn
        return carry

    jax.lax.fori_loop(0, _S, step, 0)

    na = na_ref[...]
    node = jax.lax.broadcasted_iota(jnp.int32, (_N, _B), 0)
    col_a = jax.lax.broadcasted_iota(jnp.int32, (_N, _B), 1) * _A
    inv = 1.0 / jnp.maximum(na.astype(jnp.float32), 1.0)
    sel = (node >= col_a) & (node < col_a + na)
    Mm = jnp.where(sel, inv, 0.0)
    res = jnp.dot(H_ref[...], Mm, preferred_element_type=jnp.float32)

    out_ref[...] = jnp.tanh(
        jnp.dot(Wl_ref[...], res, preferred_element_type=jnp.float32)
        + jnp.dot(Whi_ref[...], hide_ref[...], preferred_element_type=jnp.float32)
        + bl_ref[...])


def kernel(agent_obs, hideout_obs, timestep_obs, num_agents,
           W_i, Wh_i, bh_i, b_i, wc_i,
           W_f, Wh_f, bh_f, b_f, wc_f,
           W_c, Wh_c, bh_c, b_c,
           W_o, Wh_o, bh_o, b_o, wc_o,
           Wl_ag, bl_ag, Wr_ag, Wl_hi, bl_hi, Wr_hi):
    xT = jnp.transpose(agent_obs, (1, 3, 0, 2)).reshape(_S, _F, _N)
    W = jnp.concatenate([W_i, W_f, W_c, W_o], axis=1).T
    Wh = jnp.concatenate([Wh_i, Wh_f, Wh_c, Wh_o], axis=1).T
    b = jnp.concatenate([b_i + bh_i, b_f + bh_f,
                         b_c + bh_c, b_o + bh_o]).reshape(_G4, 1)
    wci = wc_i.reshape(_HID, 1)
    wcf = wc_f.reshape(_HID, 1)
    wco = wc_o.reshape(_HID, 1)
    na2 = num_agents.reshape(1, _B).astype(jnp.int32)
    hideT = hideout_obs.T
    Wl = Wl_ag.T
    Whi = Wl_hi.T
    bl = (bl_ag + bl_hi).reshape(_OUT, 1)

    out_t = pl.pallas_call(
        _gclstm_kernel,
        out_shape=jax.ShapeDtypeStruct((_OUT, _B), jnp.float32),
        scratch_shapes=[pltpu.VMEM((_HID, _N), jnp.float32),
                        pltpu.VMEM((_HID, _N), jnp.float32)],
    )(xT, hideT, na2, W, Wh, b, wci, wcf, wco, Wl, Whi, bl)
    return out_t.T


# speedup vs baseline: 3.5264x; 1.2539x over previous
import jax
import jax.numpy as jnp
from jax.experimental import pallas as pl
from jax.experimental.pallas import tpu as pltpu

_B, _S, _A, _F, _HID, _OUT = 256, 50, 16, 16, 64, 64
_N = _B * _A
_G4 = 4 * _HID
_XH = _F + _HID


def _gclstm_kernel(x_ref, hide_ref, na_ref, W2_ref, b_ref,
                   wci_ref, wcf_ref, wco_ref, Wl_ref, Whi_ref, bl_ref,
                   out_ref, XH_ref, C_ref):
    XH_ref[...] = jnp.zeros((_XH, _N), jnp.float32)
    C_ref[...] = jnp.zeros((_HID, _N), jnp.float32)
    W2 = W2_ref[...]
    b = b_ref[...]
    wci = wci_ref[...]
    wcf = wcf_ref[...]
    wco = wco_ref[...]

    def step(t, carry):
        XH_ref[0:_F] = x_ref[t]
        C = C_ref[...]
        G = jnp.dot(W2, XH_ref[...],
                    preferred_element_type=jnp.float32) + b
        gi = jnp.tanh(G[0:_HID] + wci * C) * 0.5 + 0.5
        gf = jnp.tanh(G[_HID:2 * _HID] + wcf * C) * 0.5 + 0.5
        gt = jnp.tanh(G[2 * _HID:3 * _HID])
        Cn = gf * C + gi * gt
        go = jnp.tanh(G[3 * _HID:4 * _HID] + wco * Cn) * 0.5 + 0.5
        XH_ref[_F:_XH] = go * jnp.tanh(Cn)
        C_ref[...] = Cn
        return carry

    jax.lax.fori_loop(0, _S, step, 0)

    na = na_ref[...]
    node = jax.lax.broadcasted_iota(jnp.int32, (_N, _B), 0)
    col_a = jax.lax.broadcasted_iota(jnp.int32, (_N, _B), 1) * _A
    inv = 1.0 / jnp.maximum(na.astype(jnp.float32), 1.0)
    sel = (node >= col_a) & (node < col_a + na)
    Mm = jnp.where(sel, inv, 0.0)
    res = jnp.dot(XH_ref[_F:_XH], Mm,
                  preferred_element_type=jnp.float32)

    out_ref[...] = jnp.tanh(
        jnp.dot(Wl_ref[...], res, preferred_element_type=jnp.float32)
        + jnp.dot(Whi_ref[...], hide_ref[...], preferred_element_type=jnp.float32)
        + bl_ref[...])


def kernel(agent_obs, hideout_obs, timestep_obs, num_agents,
           W_i, Wh_i, bh_i, b_i, wc_i,
           W_f, Wh_f, bh_f, b_f, wc_f,
           W_c, Wh_c, bh_c, b_c,
           W_o, Wh_o, bh_o, b_o, wc_o,
           Wl_ag, bl_ag, Wr_ag, Wl_hi, bl_hi, Wr_hi):
    xT = jnp.transpose(agent_obs, (1, 3, 0, 2)).reshape(_S, _F, _N)
    W = jnp.concatenate([W_i, W_f, W_c, W_o], axis=1).T
    Wh = jnp.concatenate([Wh_i, Wh_f, Wh_c, Wh_o], axis=1).T
    W2 = jnp.concatenate([W, Wh], axis=1)
    b = jnp.concatenate([b_i + bh_i, b_f + bh_f,
                         b_c + bh_c, b_o + bh_o]).reshape(_G4, 1)
    scale = jnp.concatenate([jnp.full((_HID,), 0.5, jnp.float32),
                             jnp.full((_HID,), 0.5, jnp.float32),
                             jnp.ones((_HID,), jnp.float32),
                             jnp.full((_HID,), 0.5, jnp.float32)]).reshape(_G4, 1)
    W2 = W2 * scale
    b = b * scale
    wci = wc_i.reshape(_HID, 1) * 0.5
    wcf = wc_f.reshape(_HID, 1) * 0.5
    wco = wc_o.reshape(_HID, 1) * 0.5
    na2 = num_agents.reshape(1, _B).astype(jnp.int32)
    hideT = hideout_obs.T
    Wl = Wl_ag.T
    Whi = Wl_hi.T
    bl = (bl_ag + bl_hi).reshape(_OUT, 1)

    out_t = pl.pallas_call(
        _gclstm_kernel,
        out_shape=jax.ShapeDtypeStruct((_OUT, _B), jnp.float32),
        scratch_shapes=[pltpu.VMEM((_XH, _N), jnp.float32),
                        pltpu.VMEM((_HID, _N), jnp.float32)],
    )(xT, hideT, na2, W2, b, wci, wcf, wco, Wl, Whi, bl)
    return out_t.T


# speedup vs baseline: 3.9080x; 1.1082x over previous
import jax
import jax.numpy as jnp
from jax.experimental import pallas as pl
from jax.experimental.pallas import tpu as pltpu

_B, _S, _A, _F, _HID, _OUT = 256, 50, 16, 16, 64, 64
_N = _B * _A
_G4 = 4 * _HID
_RH = _F
_RC = _F + _HID
_R1 = _F + 2 * _HID
_K = _R1 + 1


def _gclstm_kernel(x_ref, hide_ref, na_ref, W3_ref, wco_ref,
                   Wl_ref, Whi_ref, bl_ref, out_ref, XH_ref):
    XH_ref[...] = jnp.zeros((_K, _N), jnp.float32)
    XH_ref[_R1:_K] = jnp.ones((1, _N), jnp.float32)
    W3 = W3_ref[...]
    wco = wco_ref[...]

    def step(t, carry):
        XH_ref[0:_F] = x_ref[t]
        G = jnp.dot(W3, XH_ref[...], preferred_element_type=jnp.float32)
        gi = jnp.tanh(G[0:_HID]) * 0.5 + 0.5
        gf = jnp.tanh(G[_HID:2 * _HID]) * 0.5 + 0.5
        gt = jnp.tanh(G[2 * _HID:3 * _HID])
        C = XH_ref[_RC:_R1]
        Cn = gf * C + gi * gt
        go = jnp.tanh(G[3 * _HID:4 * _HID] + wco * Cn) * 0.5 + 0.5
        XH_ref[_RH:_RC] = go * jnp.tanh(Cn)
        XH_ref[_RC:_R1] = Cn
        return carry

    jax.lax.fori_loop(0, _S, step, 0)

    na = na_ref[...]
    node = jax.lax.broadcasted_iota(jnp.int32, (_N, _B), 0)
    col_a = jax.lax.broadcasted_iota(jnp.int32, (_N, _B), 1) * _A
    inv = 1.0 / jnp.maximum(na.astype(jnp.float32), 1.0)
    sel = (node >= col_a) & (node < col_a + na)
    Mm = jnp.where(sel, inv, 0.0)
    res = jnp.dot(XH_ref[_RH:_RC], Mm,
                  preferred_element_type=jnp.float32)

    out_ref[...] = jnp.tanh(
        jnp.dot(Wl_ref[...], res, preferred_element_type=jnp.float32)
        + jnp.dot(Whi_ref[...], hide_ref[...], preferred_element_type=jnp.float32)
        + bl_ref[...])


def kernel(agent_obs, hideout_obs, timestep_obs, num_agents,
           W_i, Wh_i, bh_i, b_i, wc_i,
           W_f, Wh_f, bh_f, b_f, wc_f,
           W_c, Wh_c, bh_c, b_c,
           W_o, Wh_o, bh_o, b_o, wc_o,
           Wl_ag, bl_ag, Wr_ag, Wl_hi, bl_hi, Wr_hi):
    xT = jnp.transpose(agent_obs, (1, 3, 0, 2)).reshape(_S, _F, _N)
    W = jnp.concatenate([W_i, W_f, W_c, W_o], axis=1).T
    Wh = jnp.concatenate([Wh_i, Wh_f, Wh_c, Wh_o], axis=1).T
    z64 = jnp.zeros((_HID, _HID), jnp.float32)
    Wc = jnp.concatenate([jnp.diag(wc_i.reshape(-1)),
                          jnp.diag(wc_f.reshape(-1)),
                          z64, z64], axis=0)
    b = jnp.concatenate([b_i + bh_i, b_f + bh_f,
                         b_c + bh_c, b_o + bh_o]).reshape(_G4, 1)
    W3 = jnp.concatenate([W, Wh, Wc, b], axis=1)
    scale = jnp.concatenate([jnp.full((_HID,), 0.5, jnp.float32),
                             jnp.full((_HID,), 0.5, jnp.float32),
                             jnp.ones((_HID,), jnp.float32),
                             jnp.full((_HID,), 0.5, jnp.float32)]).reshape(_G4, 1)
    W3 = W3 * scale
    wco = wc_o.reshape(_HID, 1) * 0.5
    na2 = num_agents.reshape(1, _B).astype(jnp.int32)
    hideT = hideout_obs.T
    Wl = Wl_ag.T
    Whi = Wl_hi.T
    bl = (bl_ag + bl_hi).reshape(_OUT, 1)

    out_t = pl.pallas_call(
        _gclstm_kernel,
        out_shape=jax.ShapeDtypeStruct((_OUT, _B), jnp.float32),
        scratch_shapes=[pltpu.VMEM((_K, _N), jnp.float32)],
    )(xT, hideT, na2, W3, wco, Wl, Whi, bl)
    return out_t.T
